# baseline (speedup 1.0000x reference)
import functools

import jax
import jax.numpy as jnp
from jax import lax
from jax.experimental import pallas as pl
from jax.experimental.pallas import tpu as pltpu
from jax.experimental.pallas import tpu_sc as plsc

_N = 10000
_NP = 10240
_D = 128
_C = 40
_G = 128
_E = 320000
_NC, _NS = 2, 16
_NW = _NC * _NS
_CH = 128
_K = 80
_NBUF = 2
_GRP = 20
_NG = _K // _GRP
_NT = _NG // 2
_EPT = _K * _CH
_EP = _NW * _EPT
_RPT = _NP // _NS
_BLK = 256
_GRID = _NP // _BLK


def _sc_degree(dstz, z1, ones1):
  mesh = plsc.VectorSubcoreMesh(core_axis_name="c", subcore_axis_name="s")

  @functools.partial(
      pl.kernel,
      out_type=jax.ShapeDtypeStruct((_NC, _NP), jnp.float32),
      mesh=mesh,
      scratch_types=[
          pltpu.VMEM((_K, _CH), jnp.int32),
          pltpu.VMEM((_CH,), jnp.float32),
          pltpu.VMEM_SHARED((_NP,), jnp.float32),
      ],
  )
  def deg_kernel(dst_hbm, z_hbm, ones_hbm, out_hbm, dst_v, ones_v, acc):
    cid = lax.axis_index("c")
    sid = lax.axis_index("s")
    wid = cid * _NS + sid
    pltpu.sync_copy(dst_hbm.at[wid], dst_v)
    pltpu.sync_copy(ones_hbm, ones_v)
    pltpu.sync_copy(z_hbm.at[pl.ds(sid * _RPT, _RPT)],
                    acc.at[pl.ds(sid * _RPT, _RPT)])
    plsc.subcore_barrier()

    def body(j, carry):
      pltpu.sync_copy(ones_v, acc.at[dst_v.at[j]], add=True)
      return carry

    lax.fori_loop(0, _K, body, 0)
    plsc.subcore_barrier()
    pltpu.sync_copy(acc.at[pl.ds(sid * _RPT, _RPT)],
                    out_hbm.at[cid].at[pl.ds(sid * _RPT, _RPT)])

  return deg_kernel(dstz, z1, ones1)


def _sc_aggregate(hp, idxp, zrows):
  mesh = plsc.VectorSubcoreMesh(core_axis_name="c", subcore_axis_name="s")

  @functools.partial(
      pl.kernel,
      out_type=jax.ShapeDtypeStruct((_NC, _NP, _D), jnp.float32),
      mesh=mesh,
      scratch_types=[
          pltpu.VMEM((2, _GRP, 2, _CH), jnp.int32),
          pltpu.VMEM((_NBUF, _CH, _D), jnp.float32),
          pltpu.VMEM_SHARED((_NP, _D), jnp.float32),
          pltpu.SemaphoreType.DMA,
          pltpu.SemaphoreType.DMA,
          pltpu.SemaphoreType.DMA,
          pltpu.SemaphoreType.DMA,
      ],
  )
  def agg_kernel(hp_hbm, idx_hbm, z_hbm, out_hbm,
                 idx_v, rows_v, acc, g0, g1, i0, i1):
    gsems = (g0, g1)
    isems = (i0, i1)
    cid = lax.axis_index("c")
    sid = lax.axis_index("s")
    wid = cid * _NS + sid
    my_idx = idx_hbm.at[wid]

    def load_group(q, buf):
      pltpu.async_copy(my_idx.at[pl.ds(q * _GRP, _GRP)], idx_v.at[buf],
                       isems[buf])

    def wait_group(buf):
      pltpu.make_async_copy(my_idx.at[pl.ds(0, _GRP)], idx_v.at[buf],
                            isems[buf]).wait()

    def issue_gather(idxbuf, row, rbuf):
      pltpu.async_copy(hp_hbm.at[idx_v.at[idxbuf].at[row].at[0]],
                       rows_v.at[rbuf], gsems[rbuf])

    def wait_gather(rbuf):
      pltpu.make_async_copy(hp_hbm.at[idx_v.at[0].at[0].at[0]],
                            rows_v.at[rbuf], gsems[rbuf]).wait()

    load_group(0, 0)
    load_group(1, 1)
    pltpu.sync_copy(z_hbm.at[pl.ds(sid * _RPT, _RPT)],
                    acc.at[pl.ds(sid * _RPT, _RPT)])
    plsc.subcore_barrier()
    wait_group(0)
    issue_gather(0, 0, 0)
    issue_gather(0, 1, 1)

    def trip(t, carry):
      for gb in range(2):
        for p in range(_GRP):
          rbuf = p % 2
          wait_gather(rbuf)
          pltpu.sync_copy(rows_v.at[rbuf],
                          acc.at[idx_v.at[gb].at[p].at[1]], add=True)
          if p == _GRP - 2:
            if gb == 0:
              wait_group(1)
            else:
              @pl.when(t == 0)
              def _():
                wait_group(0)
          if p < _GRP - 2:
            issue_gather(gb, p + 2, rbuf)
          else:
            if gb == 0:
              issue_gather(1, p - (_GRP - 2), rbuf)
            else:
              @pl.when(t == 0)
              def _():
                issue_gather(0, p - (_GRP - 2), rbuf)
        if gb == 0:
          @pl.when(t == 0)
          def _():
            load_group(2 * t + 2, 0)
        else:
          @pl.when(t == 0)
          def _():
            load_group(2 * t + 3, 1)
      return carry

    lax.fori_loop(0, _NT, trip, 0)
    plsc.subcore_barrier()
    pltpu.sync_copy(acc.at[pl.ds(sid * _RPT, _RPT)],
                    out_hbm.at[cid].at[pl.ds(sid * _RPT, _RPT)])

  return agg_kernel(hp, idxp, zrows)


def _tc_mm1(deg2, xp, W1):
  def body(deg_ref, x_ref, w_ref, h_ref, dv_ref):
    dv = lax.rsqrt(deg_ref[0] + deg_ref[1] + 1.0)
    dv_ref[...] = dv
    h_ref[...] = dv * jnp.dot(x_ref[...], w_ref[...],
                              preferred_element_type=jnp.float32)

  return pl.pallas_call(
      body,
      grid=(_GRID,),
      in_specs=[
          pl.BlockSpec((_NC, _BLK, 1), lambda i: (0, i, 0)),
          pl.BlockSpec((_BLK, _D), lambda i: (i, 0)),
          pl.BlockSpec((_D, _D), lambda i: (0, 0)),
      ],
      out_specs=[
          pl.BlockSpec((_BLK, _D), lambda i: (i, 0)),
          pl.BlockSpec((_BLK, 1), lambda i: (i, 0)),
      ],
      out_shape=[
          jax.ShapeDtypeStruct((_NP, _D), jnp.float32),
          jax.ShapeDtypeStruct((_NP, 1), jnp.float32),
      ],
  )(deg2, xp, W1)


def _tc_mm_mid(p, hp, dinv, b, W):
  def body(p_ref, hp_ref, dv_ref, b_ref, w_ref, o_ref):
    dv = dv_ref[...]
    xa = jnp.maximum(dv * (p_ref[0] + p_ref[1] + hp_ref[...]) + b_ref[...],
                     0.0)
    o_ref[...] = dv * jnp.dot(xa, w_ref[...],
                              preferred_element_type=jnp.float32)

  return pl.pallas_call(
      body,
      grid=(_GRID,),
      in_specs=[
          pl.BlockSpec((_NC, _BLK, _D), lambda i: (0, i, 0)),
          pl.BlockSpec((_BLK, _D), lambda i: (i, 0)),
          pl.BlockSpec((_BLK, 1), lambda i: (i, 0)),
          pl.BlockSpec((1, _D), lambda i: (0, 0)),
          pl.BlockSpec((_D, _D), lambda i: (0, 0)),
      ],
      out_specs=pl.BlockSpec((_BLK, _D), lambda i: (i, 0)),
      out_shape=jax.ShapeDtypeStruct((_NP, _D), jnp.float32),
  )(p, hp, dinv, b, W)


def _tc_final(p, hp, dinv, b3, batchc, Wl, bl):
  def body(p_ref, hp_ref, dv_ref, b_ref, bt_ref, wl_ref, bl_ref, o_ref,
           acc, cnt):
    i = pl.program_id(0)

    @pl.when(i == 0)
    def _():
      acc[...] = jnp.zeros_like(acc)
      cnt[...] = jnp.zeros_like(cnt)

    h3 = dv_ref[...] * (p_ref[0] + p_ref[1] + hp_ref[...]) + b_ref[...]
    onehot = (lax.broadcasted_iota(jnp.int32, (_BLK, _G), 1)
              == bt_ref[...]).astype(jnp.float32)
    acc[...] += lax.dot_general(onehot, h3, (((0,), (0,)), ((), ())),
                                preferred_element_type=jnp.float32)
    cnt[...] += lax.dot_general(onehot, jnp.ones((_BLK, 1), jnp.float32),
                                (((0,), (0,)), ((), ())),
                                preferred_element_type=jnp.float32)

    @pl.when(i == _GRID - 1)
    def _():
      pooled = acc[...] / jnp.maximum(cnt[...], 1.0)
      o_ref[...] = jnp.dot(pooled, wl_ref[...],
                           preferred_element_type=jnp.float32) + bl_ref[...]

  return pl.pallas_call(
      body,
      grid=(_GRID,),
      in_specs=[
          pl.BlockSpec((_NC, _BLK, _D), lambda i: (0, i, 0)),
          pl.BlockSpec((_BLK, _D), lambda i: (i, 0)),
          pl.BlockSpec((_BLK, 1), lambda i: (i, 0)),
          pl.BlockSpec((1, _D), lambda i: (0, 0)),
          pl.BlockSpec((_BLK, 1), lambda i: (i, 0)),
          pl.BlockSpec((_D, _C), lambda i: (0, 0)),
          pl.BlockSpec((1, _C), lambda i: (0, 0)),
      ],
      out_specs=pl.BlockSpec((_G, _C), lambda i: (0, 0)),
      out_shape=jax.ShapeDtypeStruct((_G, _C), jnp.float32),
      scratch_shapes=[
          pltpu.VMEM((_G, _D), jnp.float32),
          pltpu.VMEM((_G, 1), jnp.float32),
      ],
  )(p, hp, dinv, b3, batchc, Wl, bl)


def kernel(x, edge_index, batch, W1, b1, W2, b2, W3, b3, Wl, bl):
  xp = jnp.pad(x, ((0, _NP - _N), (0, 0)))
  src = edge_index[0]
  dst = edge_index[1]
  pad = _EP - _E
  srcz = jnp.concatenate([src, jnp.zeros((pad,), jnp.int32)]).reshape(
      _NW, _K, _CH)
  pdst = _N + (jnp.arange(pad, dtype=jnp.int32) % (_NP - _N))
  dstz = jnp.concatenate([dst, pdst]).reshape(_NW, _K, _CH)
  idxp = jnp.stack([srcz, dstz], axis=2)
  zrows = jnp.zeros((_NP, _D), jnp.float32)
  z1 = jnp.zeros((_NP,), jnp.float32)
  ones1 = jnp.ones((_CH,), jnp.float32)
  batchc = jnp.pad(batch, (0, _NP - _N), constant_values=_G).reshape(_NP, 1)

  deg = _sc_degree(dstz, z1, ones1)
  deg2 = deg.reshape(_NC, _NP, 1)
  h1, dinv = _tc_mm1(deg2, xp, W1)
  p1 = _sc_aggregate(h1, idxp, zrows)
  h2 = _tc_mm_mid(p1, h1, dinv, b1.reshape(1, _D), W2)
  p2 = _sc_aggregate(h2, idxp, zrows)
  h3 = _tc_mm_mid(p2, h2, dinv, b2.reshape(1, _D), W3)
  p3 = _sc_aggregate(h3, idxp, zrows)
  return _tc_final(p3, h3, dinv, b3.reshape(1, _D), batchc, Wl,
                   bl.reshape(1, _C))


# speedup vs baseline: 3.0641x; 3.0641x over previous
import functools

import jax
import jax.numpy as jnp
from jax import lax
from jax.experimental import pallas as pl
from jax.experimental.pallas import tpu as pltpu
from jax.experimental.pallas import tpu_sc as plsc

_N = 10000
_NP = 10240
_D = 128
_C = 40
_G = 128
_E = 320000
_NC, _NS = 2, 16
_NW = _NC * _NS
_CH = 128
_K = 80
_NBUF = 2
_GRP = 20
_NG = _K // _GRP
_NT = _NG // 2
_EPT = _K * _CH
_EP = _NW * _EPT
_RPT = _NP // _NS
_BLK = 256
_GRID = _NP // _BLK


def _sc_degree(dstz, z1, ones1):
  mesh = plsc.VectorSubcoreMesh(core_axis_name="c", subcore_axis_name="s")

  @functools.partial(
      pl.kernel,
      out_type=jax.ShapeDtypeStruct((_NC, _NP), jnp.float32),
      mesh=mesh,
      scratch_types=[
          pltpu.VMEM((_K, _CH), jnp.int32),
          pltpu.VMEM((_CH,), jnp.float32),
          pltpu.VMEM_SHARED((_NP,), jnp.float32),
      ],
  )
  def deg_kernel(dst_hbm, z_hbm, ones_hbm, out_hbm, dst_v, ones_v, acc):
    cid = lax.axis_index("c")
    sid = lax.axis_index("s")
    wid = cid * _NS + sid
    pltpu.sync_copy(dst_hbm.at[wid], dst_v)
    pltpu.sync_copy(ones_hbm, ones_v)
    pltpu.sync_copy(z_hbm.at[pl.ds(sid * _RPT, _RPT)],
                    acc.at[pl.ds(sid * _RPT, _RPT)])
    plsc.subcore_barrier()

    def body(j, carry):
      pltpu.sync_copy(ones_v, acc.at[dst_v.at[j]], add=True)
      return carry

    lax.fori_loop(0, _K, body, 0)
    plsc.subcore_barrier()
    pltpu.sync_copy(acc.at[pl.ds(sid * _RPT, _RPT)],
                    out_hbm.at[cid].at[pl.ds(sid * _RPT, _RPT)])

  return deg_kernel(dstz, z1, ones1)


def _sc_aggregate(hp, idxp, zrows):
  mesh = plsc.VectorSubcoreMesh(core_axis_name="c", subcore_axis_name="s")

  @functools.partial(
      pl.kernel,
      out_type=jax.ShapeDtypeStruct((_NC, _NP, _D), jnp.float32),
      mesh=mesh,
      scratch_types=[
          pltpu.VMEM((2, _GRP, 2, _CH), jnp.int32),
          pltpu.VMEM((_NBUF, _CH, _D), jnp.float32),
          pltpu.VMEM_SHARED((_NP, _D), jnp.float32),
          pltpu.SemaphoreType.DMA,
          pltpu.SemaphoreType.DMA,
          pltpu.SemaphoreType.DMA,
          pltpu.SemaphoreType.DMA,
      ],
  )
  def agg_kernel(hp_hbm, idx_hbm, z_hbm, out_hbm,
                 idx_v, rows_v, acc, g0, g1, i0, i1):
    gsems = (g0, g1)
    isems = (i0, i1)
    cid = lax.axis_index("c")
    sid = lax.axis_index("s")
    wid = cid * _NS + sid
    my_idx = idx_hbm.at[wid]

    def load_group(q, buf):
      pltpu.async_copy(my_idx.at[pl.ds(q * _GRP, _GRP)], idx_v.at[buf],
                       isems[buf])

    def wait_group(buf):
      pltpu.make_async_copy(my_idx.at[pl.ds(0, _GRP)], idx_v.at[buf],
                            isems[buf]).wait()

    def issue_gather(idxbuf, row, rbuf):
      pltpu.async_copy(hp_hbm.at[idx_v.at[idxbuf].at[row].at[0]],
                       rows_v.at[rbuf], gsems[rbuf])

    def wait_gather(rbuf):
      pltpu.make_async_copy(hp_hbm.at[idx_v.at[0].at[0].at[0]],
                            rows_v.at[rbuf], gsems[rbuf]).wait()

    load_group(0, 0)
    load_group(1, 1)
    pltpu.sync_copy(z_hbm.at[pl.ds(sid * _RPT, _RPT)],
                    acc.at[pl.ds(sid * _RPT, _RPT)])
    plsc.subcore_barrier()
    wait_group(0)
    issue_gather(0, 0, 0)
    issue_gather(0, 1, 1)

    def trip(t, carry):
      for gb in range(2):
        for p in range(_GRP):
          rbuf = p % 2
          wait_gather(rbuf)
          pltpu.sync_copy(rows_v.at[rbuf],
                          acc.at[idx_v.at[gb].at[p].at[1]], add=True)
          if p == _GRP - 2:
            if gb == 0:
              wait_group(1)
            else:
              @pl.when(t == 0)
              def _():
                wait_group(0)
          if p < _GRP - 2:
            issue_gather(gb, p + 2, rbuf)
          else:
            if gb == 0:
              issue_gather(1, p - (_GRP - 2), rbuf)
            else:
              @pl.when(t == 0)
              def _():
                issue_gather(0, p - (_GRP - 2), rbuf)
        if gb == 0:
          @pl.when(t == 0)
          def _():
            load_group(2 * t + 2, 0)
        else:
          @pl.when(t == 0)
          def _():
            load_group(2 * t + 3, 1)
      return carry

    lax.fori_loop(0, _NT, trip, 0)
    plsc.subcore_barrier()
    pltpu.sync_copy(acc.at[pl.ds(sid * _RPT, _RPT)],
                    out_hbm.at[cid].at[pl.ds(sid * _RPT, _RPT)])

  return agg_kernel(hp, idxp, zrows)


def _tc_mm1(deg2, xp, W1):
  def body(deg_ref, x_ref, w_ref, h_ref, dv_ref):
    dv = lax.rsqrt(deg_ref[0] + deg_ref[1] + 1.0)
    dv_ref[...] = dv
    h_ref[...] = dv * jnp.dot(x_ref[...], w_ref[...],
                              preferred_element_type=jnp.float32)

  return pl.pallas_call(
      body,
      grid=(_GRID,),
      in_specs=[
          pl.BlockSpec((_NC, _BLK, 1), lambda i: (0, i, 0)),
          pl.BlockSpec((_BLK, _D), lambda i: (i, 0)),
          pl.BlockSpec((_D, _D), lambda i: (0, 0)),
      ],
      out_specs=[
          pl.BlockSpec((_BLK, _D), lambda i: (i, 0)),
          pl.BlockSpec((_BLK, 1), lambda i: (i, 0)),
      ],
      out_shape=[
          jax.ShapeDtypeStruct((_NP, _D), jnp.float32),
          jax.ShapeDtypeStruct((_NP, 1), jnp.float32),
      ],
  )(deg2, xp, W1)


def _tc_mm_mid(p, hp, dinv, b, W):
  def body(p_ref, hp_ref, dv_ref, b_ref, w_ref, o_ref):
    dv = dv_ref[...]
    xa = jnp.maximum(dv * (p_ref[0] + p_ref[1] + hp_ref[...]) + b_ref[...],
                     0.0)
    o_ref[...] = dv * jnp.dot(xa, w_ref[...],
                              preferred_element_type=jnp.float32)

  return pl.pallas_call(
      body,
      grid=(_GRID,),
      in_specs=[
          pl.BlockSpec((_NC, _BLK, _D), lambda i: (0, i, 0)),
          pl.BlockSpec((_BLK, _D), lambda i: (i, 0)),
          pl.BlockSpec((_BLK, 1), lambda i: (i, 0)),
          pl.BlockSpec((1, _D), lambda i: (0, 0)),
          pl.BlockSpec((_D, _D), lambda i: (0, 0)),
      ],
      out_specs=pl.BlockSpec((_BLK, _D), lambda i: (i, 0)),
      out_shape=jax.ShapeDtypeStruct((_NP, _D), jnp.float32),
  )(p, hp, dinv, b, W)


def _tc_final(p, hp, dinv, b3, batchc, Wl, bl):
  def body(p_ref, hp_ref, dv_ref, b_ref, bt_ref, wl_ref, bl_ref, o_ref,
           acc, cnt):
    i = pl.program_id(0)

    @pl.when(i == 0)
    def _():
      acc[...] = jnp.zeros_like(acc)
      cnt[...] = jnp.zeros_like(cnt)

    h3 = dv_ref[...] * (p_ref[0] + p_ref[1] + hp_ref[...]) + b_ref[...]
    onehot = (lax.broadcasted_iota(jnp.int32, (_BLK, _G), 1)
              == bt_ref[...]).astype(jnp.float32)
    acc[...] += lax.dot_general(onehot, h3, (((0,), (0,)), ((), ())),
                                preferred_element_type=jnp.float32)
    cnt[...] += lax.dot_general(onehot, jnp.ones((_BLK, 1), jnp.float32),
                                (((0,), (0,)), ((), ())),
                                preferred_element_type=jnp.float32)

    @pl.when(i == _GRID - 1)
    def _():
      pooled = acc[...] / jnp.maximum(cnt[...], 1.0)
      o_ref[...] = jnp.dot(pooled, wl_ref[...],
                           preferred_element_type=jnp.float32) + bl_ref[...]

  return pl.pallas_call(
      body,
      grid=(_GRID,),
      in_specs=[
          pl.BlockSpec((_NC, _BLK, _D), lambda i: (0, i, 0)),
          pl.BlockSpec((_BLK, _D), lambda i: (i, 0)),
          pl.BlockSpec((_BLK, 1), lambda i: (i, 0)),
          pl.BlockSpec((1, _D), lambda i: (0, 0)),
          pl.BlockSpec((_BLK, 1), lambda i: (i, 0)),
          pl.BlockSpec((_D, _C), lambda i: (0, 0)),
          pl.BlockSpec((1, _C), lambda i: (0, 0)),
      ],
      out_specs=pl.BlockSpec((_G, _C), lambda i: (0, 0)),
      out_shape=jax.ShapeDtypeStruct((_G, _C), jnp.float32),
      scratch_shapes=[
          pltpu.VMEM((_G, _D), jnp.float32),
          pltpu.VMEM((_G, 1), jnp.float32),
      ],
  )(p, hp, dinv, b3, batchc, Wl, bl)


def kernel(x, edge_index, batch, W1, b1, W2, b2, W3, b3, Wl, bl):
  xp = jnp.pad(x, ((0, _NP - _N), (0, 0)))
  src = edge_index[0]
  dst = edge_index[1]
  pad = _EP - _E
  ppt = pad // _NW
  rpt_e = _E // _NW
  pcyc = _N + (jnp.arange(pad, dtype=jnp.int32) % (_NP - _N))
  srcz = jnp.concatenate(
      [src.reshape(_NW, rpt_e), pcyc.reshape(_NW, ppt)], axis=1
  ).reshape(_NW, _K, _CH)
  dstz = jnp.concatenate(
      [dst.reshape(_NW, rpt_e), pcyc.reshape(_NW, ppt)], axis=1
  ).reshape(_NW, _K, _CH)
  idxp = jnp.stack([srcz, dstz], axis=2)
  zrows = jnp.zeros((_NP, _D), jnp.float32)
  z1 = jnp.zeros((_NP,), jnp.float32)
  ones1 = jnp.ones((_CH,), jnp.float32)
  batchc = jnp.pad(batch, (0, _NP - _N), constant_values=_G).reshape(_NP, 1)

  deg = _sc_degree(dstz, z1, ones1)
  deg2 = deg.reshape(_NC, _NP, 1)
  h1, dinv = _tc_mm1(deg2, xp, W1)
  p1 = _sc_aggregate(h1, idxp, zrows)
  h2 = _tc_mm_mid(p1, h1, dinv, b1.reshape(1, _D), W2)
  p2 = _sc_aggregate(h2, idxp, zrows)
  h3 = _tc_mm_mid(p2, h2, dinv, b2.reshape(1, _D), W3)
  p3 = _sc_aggregate(h3, idxp, zrows)
  return _tc_final(p3, h3, dinv, b3.reshape(1, _D), batchc, Wl,
                   bl.reshape(1, _C))


# speedup vs baseline: 3.0914x; 1.0089x over previous
import functools

import jax
import jax.numpy as jnp
from jax import lax
from jax.experimental import pallas as pl
from jax.experimental.pallas import tpu as pltpu
from jax.experimental.pallas import tpu_sc as plsc

_N = 10000
_NP = 10240
_D = 128
_C = 40
_G = 128
_E = 320000
_NC, _NS = 2, 16
_NW = _NC * _NS
_CH = 128
_K = 80
_NBUF = 2
_GRP = 20
_NG = _K // _GRP
_NT = _NG // 2
_EPT = _K * _CH
_EP = _NW * _EPT
_RPT = _NP // _NS
_BLK = 256
_GRID = _NP // _BLK


def _sc_degree(dstz, z1, ones1):
  mesh = plsc.VectorSubcoreMesh(core_axis_name="c", subcore_axis_name="s")

  @functools.partial(
      pl.kernel,
      out_type=jax.ShapeDtypeStruct((_NC, _NP), jnp.float32),
      mesh=mesh,
      scratch_types=[
          pltpu.VMEM((_K, _CH), jnp.int32),
          pltpu.VMEM((_CH,), jnp.float32),
          pltpu.VMEM_SHARED((_NP,), jnp.float32),
      ],
  )
  def deg_kernel(dst_hbm, z_hbm, ones_hbm, out_hbm, dst_v, ones_v, acc):
    cid = lax.axis_index("c")
    sid = lax.axis_index("s")
    wid = cid * _NS + sid
    pltpu.sync_copy(dst_hbm.at[wid], dst_v)
    pltpu.sync_copy(ones_hbm, ones_v)
    pltpu.sync_copy(z_hbm.at[pl.ds(sid * _RPT, _RPT)],
                    acc.at[pl.ds(sid * _RPT, _RPT)])
    plsc.subcore_barrier()

    def body(j, carry):
      pltpu.sync_copy(ones_v, acc.at[dst_v.at[j]], add=True)
      return carry

    lax.fori_loop(0, _K, body, 0)
    plsc.subcore_barrier()
    pltpu.sync_copy(acc.at[pl.ds(sid * _RPT, _RPT)],
                    out_hbm.at[cid].at[pl.ds(sid * _RPT, _RPT)])

  return deg_kernel(dstz, z1, ones1)


def _sc_aggregate(hp, idxp, zrows):
  mesh = plsc.VectorSubcoreMesh(core_axis_name="c", subcore_axis_name="s")

  @functools.partial(
      pl.kernel,
      out_type=jax.ShapeDtypeStruct((_NC, _NP, _D), jnp.float32),
      mesh=mesh,
      scratch_types=[
          pltpu.VMEM((2, _GRP, 2, _CH), jnp.int32),
          pltpu.VMEM((_NBUF, _CH, _D), jnp.float32),
          pltpu.VMEM_SHARED((_NP, _D), jnp.float32),
          pltpu.SemaphoreType.DMA,
          pltpu.SemaphoreType.DMA,
          pltpu.SemaphoreType.DMA,
          pltpu.SemaphoreType.DMA,
      ],
  )
  def agg_kernel(hp_hbm, idx_hbm, z_hbm, out_hbm,
                 idx_v, rows_v, acc, g0, g1, i0, i1):
    gsems = (g0, g1)
    isems = (i0, i1)
    cid = lax.axis_index("c")
    sid = lax.axis_index("s")
    wid = cid * _NS + sid
    my_idx = idx_hbm.at[wid]

    def load_group(q, buf):
      pltpu.async_copy(my_idx.at[pl.ds(q * _GRP, _GRP)], idx_v.at[buf],
                       isems[buf])

    def wait_group(buf):
      pltpu.make_async_copy(my_idx.at[pl.ds(0, _GRP)], idx_v.at[buf],
                            isems[buf]).wait()

    def issue_gather(idxbuf, row, rbuf):
      pltpu.async_copy(hp_hbm.at[idx_v.at[idxbuf].at[row].at[0]],
                       rows_v.at[rbuf], gsems[rbuf])

    def wait_gather(rbuf):
      pltpu.make_async_copy(hp_hbm.at[idx_v.at[0].at[0].at[0]],
                            rows_v.at[rbuf], gsems[rbuf]).wait()

    load_group(0, 0)
    load_group(1, 1)

    @pl.when(cid == 0)
    def _():
      pltpu.sync_copy(hp_hbm.at[pl.ds(sid * _RPT, _RPT)],
                      acc.at[pl.ds(sid * _RPT, _RPT)])

    @pl.when(cid != 0)
    def _():
      pltpu.sync_copy(z_hbm.at[pl.ds(sid * _RPT, _RPT)],
                      acc.at[pl.ds(sid * _RPT, _RPT)])

    plsc.subcore_barrier()
    wait_group(0)
    issue_gather(0, 0, 0)
    issue_gather(0, 1, 1)

    def trip(t, carry):
      for gb in range(2):
        for p in range(_GRP):
          rbuf = p % 2
          wait_gather(rbuf)
          pltpu.sync_copy(rows_v.at[rbuf],
                          acc.at[idx_v.at[gb].at[p].at[1]], add=True)
          if p == _GRP - 2:
            if gb == 0:
              wait_group(1)
            else:
              @pl.when(t == 0)
              def _():
                wait_group(0)
          if p < _GRP - 2:
            issue_gather(gb, p + 2, rbuf)
          else:
            if gb == 0:
              issue_gather(1, p - (_GRP - 2), rbuf)
            else:
              @pl.when(t == 0)
              def _():
                issue_gather(0, p - (_GRP - 2), rbuf)
        if gb == 0:
          @pl.when(t == 0)
          def _():
            load_group(2 * t + 2, 0)
        else:
          @pl.when(t == 0)
          def _():
            load_group(2 * t + 3, 1)
      return carry

    lax.fori_loop(0, _NT, trip, 0)
    plsc.subcore_barrier()
    pltpu.sync_copy(acc.at[pl.ds(sid * _RPT, _RPT)],
                    out_hbm.at[cid].at[pl.ds(sid * _RPT, _RPT)])

  return agg_kernel(hp, idxp, zrows)


def _tc_mm1(deg2, xp, W1):
  def body(deg_ref, x_ref, w_ref, h_ref, dv_ref):
    dv = lax.rsqrt(deg_ref[0] + deg_ref[1] + 1.0)
    dv_ref[...] = dv
    h_ref[...] = dv * jnp.dot(x_ref[...], w_ref[...],
                              preferred_element_type=jnp.float32)

  return pl.pallas_call(
      body,
      grid=(_GRID,),
      in_specs=[
          pl.BlockSpec((_NC, _BLK, 1), lambda i: (0, i, 0)),
          pl.BlockSpec((_BLK, _D), lambda i: (i, 0)),
          pl.BlockSpec((_D, _D), lambda i: (0, 0)),
      ],
      out_specs=[
          pl.BlockSpec((_BLK, _D), lambda i: (i, 0)),
          pl.BlockSpec((_BLK, 1), lambda i: (i, 0)),
      ],
      out_shape=[
          jax.ShapeDtypeStruct((_NP, _D), jnp.float32),
          jax.ShapeDtypeStruct((_NP, 1), jnp.float32),
      ],
  )(deg2, xp, W1)


def _tc_mm_mid(p, dinv, b, W):
  def body(p_ref, dv_ref, b_ref, w_ref, o_ref):
    dv = dv_ref[...]
    xa = jnp.maximum(dv * (p_ref[0] + p_ref[1]) + b_ref[...], 0.0)
    o_ref[...] = dv * jnp.dot(xa, w_ref[...],
                              preferred_element_type=jnp.float32)

  return pl.pallas_call(
      body,
      grid=(_GRID,),
      in_specs=[
          pl.BlockSpec((_NC, _BLK, _D), lambda i: (0, i, 0)),
          pl.BlockSpec((_BLK, 1), lambda i: (i, 0)),
          pl.BlockSpec((1, _D), lambda i: (0, 0)),
          pl.BlockSpec((_D, _D), lambda i: (0, 0)),
      ],
      out_specs=pl.BlockSpec((_BLK, _D), lambda i: (i, 0)),
      out_shape=jax.ShapeDtypeStruct((_NP, _D), jnp.float32),
  )(p, dinv, b, W)


def _tc_final(p, dinv, b3, batchc, Wl, bl):
  def body(p_ref, dv_ref, b_ref, bt_ref, wl_ref, bl_ref, o_ref,
           acc, cnt):
    i = pl.program_id(0)

    @pl.when(i == 0)
    def _():
      acc[...] = jnp.zeros_like(acc)
      cnt[...] = jnp.zeros_like(cnt)

    h3 = dv_ref[...] * (p_ref[0] + p_ref[1]) + b_ref[...]
    onehot = (lax.broadcasted_iota(jnp.int32, (_BLK, _G), 1)
              == bt_ref[...]).astype(jnp.float32)
    acc[...] += lax.dot_general(onehot, h3, (((0,), (0,)), ((), ())),
                                preferred_element_type=jnp.float32)
    cnt[...] += lax.dot_general(onehot, jnp.ones((_BLK, 1), jnp.float32),
                                (((0,), (0,)), ((), ())),
                                preferred_element_type=jnp.float32)

    @pl.when(i == _GRID - 1)
    def _():
      pooled = acc[...] / jnp.maximum(cnt[...], 1.0)
      o_ref[...] = jnp.dot(pooled, wl_ref[...],
                           preferred_element_type=jnp.float32) + bl_ref[...]

  return pl.pallas_call(
      body,
      grid=(_GRID,),
      in_specs=[
          pl.BlockSpec((_NC, _BLK, _D), lambda i: (0, i, 0)),
          pl.BlockSpec((_BLK, 1), lambda i: (i, 0)),
          pl.BlockSpec((1, _D), lambda i: (0, 0)),
          pl.BlockSpec((_BLK, 1), lambda i: (i, 0)),
          pl.BlockSpec((_D, _C), lambda i: (0, 0)),
          pl.BlockSpec((1, _C), lambda i: (0, 0)),
      ],
      out_specs=pl.BlockSpec((_G, _C), lambda i: (0, 0)),
      out_shape=jax.ShapeDtypeStruct((_G, _C), jnp.float32),
      scratch_shapes=[
          pltpu.VMEM((_G, _D), jnp.float32),
          pltpu.VMEM((_G, 1), jnp.float32),
      ],
  )(p, dinv, b3, batchc, Wl, bl)


def kernel(x, edge_index, batch, W1, b1, W2, b2, W3, b3, Wl, bl):
  xp = jnp.pad(x, ((0, _NP - _N), (0, 0)))
  src = edge_index[0]
  dst = edge_index[1]
  pad = _EP - _E
  ppt = pad // _NW
  rpt_e = _E // _NW
  pcyc = _N + (jnp.arange(pad, dtype=jnp.int32) % (_NP - _N))
  srcz = jnp.concatenate(
      [src.reshape(_NW, rpt_e), pcyc.reshape(_NW, ppt)], axis=1
  ).reshape(_NW, _K, _CH)
  dstz = jnp.concatenate(
      [dst.reshape(_NW, rpt_e), pcyc.reshape(_NW, ppt)], axis=1
  ).reshape(_NW, _K, _CH)
  idxp = jnp.stack([srcz, dstz], axis=2)
  zrows = jnp.zeros((_NP, _D), jnp.float32)
  z1 = jnp.zeros((_NP,), jnp.float32)
  ones1 = jnp.ones((_CH,), jnp.float32)
  batchc = jnp.pad(batch, (0, _NP - _N), constant_values=_G).reshape(_NP, 1)

  deg = _sc_degree(dstz, z1, ones1)
  deg2 = deg.reshape(_NC, _NP, 1)
  h1, dinv = _tc_mm1(deg2, xp, W1)
  p1 = _sc_aggregate(h1, idxp, zrows)
  h2 = _tc_mm_mid(p1, dinv, b1.reshape(1, _D), W2)
  p2 = _sc_aggregate(h2, idxp, zrows)
  h3 = _tc_mm_mid(p2, dinv, b2.reshape(1, _D), W3)
  p3 = _sc_aggregate(h3, idxp, zrows)
  return _tc_final(p3, dinv, b3.reshape(1, _D), batchc, Wl,
                   bl.reshape(1, _C))
